# baseline (speedup 1.0000x reference)
import functools

import jax
import jax.numpy as jnp
from jax import lax
from jax.experimental import pallas as pl
from jax.experimental.pallas import tpu as pltpu
from jax.experimental.pallas import tpu_sc as plsc

B = 16384
D = 32
L = 16
NC = 2
NS = 16
NW = NC * NS
BPW = B // NW
NCHUNK = 4
CH = BPW // NCHUNK
NGROUP = BPW // L


def _sc_body(uidx_hbm, pidx_hbm, uemb_hbm, pemb_hbm, pbias_hbm,
             umask_hbm, pmask_hbm,
             score_hbm, concat_hbm,
             uidx_v, pidx_v, urows_v, prows_v, bias_v,
             umask_v, pmask_v, concat_v, score_v, tbuf_v, sem):
  c = lax.axis_index("c")
  s = lax.axis_index("s")
  wid = s * NC + c
  base = wid * BPW

  pltpu.sync_copy(uidx_hbm.at[pl.ds(wid * NCHUNK, NCHUNK)], uidx_v)
  pltpu.sync_copy(pidx_hbm.at[pl.ds(wid * NCHUNK, NCHUNK)], pidx_v)
  pltpu.sync_copy(umask_hbm, umask_v)
  pltpu.sync_copy(pmask_hbm, pmask_v)

  copies = []
  for j in range(NCHUNK):
    copies.append(pltpu.async_copy(
        uemb_hbm.at[uidx_v.at[j]], urows_v.at[pl.ds(j * CH, CH)], sem))
    copies.append(pltpu.async_copy(
        pemb_hbm.at[pidx_v.at[j]], prows_v.at[pl.ds(j * CH, CH)], sem))
    copies.append(pltpu.async_copy(
        pbias_hbm.at[pidx_v.at[j]], bias_v.at[pl.ds(j * CH, CH)], sem))
  for cp in copies:
    cp.wait()

  um0 = umask_v[pl.ds(0, L)]
  um1 = umask_v[pl.ds(L, L)]
  pm0 = pmask_v[pl.ds(0, L)]
  pm1 = pmask_v[pl.ds(L, L)]
  lane17 = lax.iota(jnp.int32, L) * (L + 1)

  def group(g, carry):
    for r in range(L):
      i = g * L + r
      u0 = urows_v[i, pl.ds(0, L)] * um0
      u1 = urows_v[i, pl.ds(L, L)] * um1
      p0 = prows_v[i, pl.ds(0, L)] * pm0
      p1 = prows_v[i, pl.ds(L, L)] * pm1
      concat_v[i, pl.ds(0, L)] = u0
      concat_v[i, pl.ds(L, L)] = u1
      concat_v[i, pl.ds(2 * L, L)] = p0
      concat_v[i, pl.ds(3 * L, L)] = p1
      t = u0 * p0 + u1 * p1
      plsc.store_scatter(tbuf_v, [lane17 + r], t)
    acc = tbuf_v[pl.ds(0, L)]
    for k in range(1, L):
      acc = acc + tbuf_v[pl.ds(k * (L + 1), L)]
    score_v[pl.ds(g * L, L)] = acc + bias_v[pl.ds(g * L, L)]
    return carry

  lax.fori_loop(0, NGROUP, group, 0, unroll=False)

  pltpu.sync_copy(score_v, score_hbm.at[pl.ds(base, BPW)])
  pltpu.sync_copy(concat_v, concat_hbm.at[pl.ds(base, BPW)])


@jax.jit
def _mvem_sc(uidx2, pidx2, user_emb, product_emb, product_bias,
             user_emb_mask, product_emb_mask):
  mesh = plsc.VectorSubcoreMesh(
      core_axis_name="c", subcore_axis_name="s", num_cores=NC,
      num_subcores=NS)
  run = pl.kernel(
      _sc_body,
      out_type=(jax.ShapeDtypeStruct((B,), jnp.float32),
                jax.ShapeDtypeStruct((B, 2 * D), jnp.float32)),
      mesh=mesh,
      scratch_types=[
          pltpu.VMEM((NCHUNK, CH), jnp.int32),
          pltpu.VMEM((NCHUNK, CH), jnp.int32),
          pltpu.VMEM((BPW, D), jnp.float32),
          pltpu.VMEM((BPW, D), jnp.float32),
          pltpu.VMEM((BPW,), jnp.float32),
          pltpu.VMEM((D,), jnp.float32),
          pltpu.VMEM((D,), jnp.float32),
          pltpu.VMEM((BPW, 2 * D), jnp.float32),
          pltpu.VMEM((BPW,), jnp.float32),
          pltpu.VMEM((L * (L + 1),), jnp.float32),
          pltpu.SemaphoreType.DMA,
      ],
      compiler_params=pltpu.CompilerParams(
          needs_layout_passes=False, use_tc_tiling_on_sc=False),
  )
  return run(uidx2, pidx2, user_emb, product_emb, product_bias,
             user_emb_mask, product_emb_mask)


def kernel(user_idx, product_idx, user_emb, product_emb, product_bias,
           user_emb_mask, product_emb_mask):
  uidx2 = user_idx.astype(jnp.int32).reshape(NW * NCHUNK, CH)
  pidx2 = product_idx.astype(jnp.int32).reshape(NW * NCHUNK, CH)
  score, concat = _mvem_sc(uidx2, pidx2, user_emb, product_emb,
                           product_bias, user_emb_mask, product_emb_mask)
  return score, concat


# speedup vs baseline: 1.4773x; 1.4773x over previous
import jax
import jax.numpy as jnp
from jax import lax
from jax.experimental import pallas as pl
from jax.experimental.pallas import tpu as pltpu
from jax.experimental.pallas import tpu_sc as plsc

B = 16384
D = 32
NC = 2
NS = 16
NW = NC * NS
BPW = B // NW
L = 16
NCH = 8
CH = BPW // NCH


def _sc_body(uidx_hbm, pidx_hbm, uemb_hbm, pemb_hbm, pbias_hbm,
             umask_hbm, pmask_hbm,
             score_hbm, concat_hbm,
             uidx_v, pidx_v, ubuf_v, pbuf_v,
             bias_v, umask_v, pmask_v, concat_v, score_v, tbuf_v,
             sem_u, sem_p, sem_b):
  c = lax.axis_index("c")
  s = lax.axis_index("s")
  wid = s * NC + c
  base = wid * BPW

  pltpu.sync_copy(uidx_hbm.at[pl.ds(base, BPW)], uidx_v)
  pltpu.sync_copy(pidx_hbm.at[pl.ds(base, BPW)], pidx_v)
  pltpu.sync_copy(umask_hbm, umask_v)
  pltpu.sync_copy(pmask_hbm, pmask_v)

  bias_cp = pltpu.async_copy(pbias_hbm.at[pidx_v], bias_v, sem_b)

  lane = lax.iota(jnp.int32, L)
  zero16 = jnp.zeros((L,), jnp.int32)

  def fire(j):
    def row_fetch(i, carry):
      off = j * CH + (i // L) * L
      sel = lane == (i % L)
      ur = jnp.sum(jnp.where(sel, uidx_v[pl.ds(off, L)], zero16))
      pr = jnp.sum(jnp.where(sel, pidx_v[pl.ds(off, L)], zero16))
      pltpu.async_copy(uemb_hbm.at[ur], ubuf_v.at[j % 2, i], sem_u)
      pltpu.async_copy(pemb_hbm.at[pr], pbuf_v.at[j % 2, i], sem_p)
      return carry
    lax.fori_loop(0, CH, row_fetch, 0)

  def drain(j):
    pltpu.make_async_copy(uemb_hbm.at[pl.ds(0, CH)], ubuf_v.at[j % 2],
                          sem_u).wait()
    pltpu.make_async_copy(pemb_hbm.at[pl.ds(0, CH)], pbuf_v.at[j % 2],
                          sem_p).wait()

  um0 = umask_v[pl.ds(0, L)]
  um1 = umask_v[pl.ds(L, L)]
  pm0 = pmask_v[pl.ds(0, L)]
  pm1 = pmask_v[pl.ds(L, L)]
  lane17 = lax.iota(jnp.int32, L) * (L + 1)

  fire(0)
  bias_cp.wait()
  for j in range(NCH):
    if j + 1 < NCH:
      fire(j + 1)
    drain(j)
    ub = ubuf_v.at[j % 2]
    pb = pbuf_v.at[j % 2]

    def group(g, carry, ub=ub, pb=pb, j=j):
      for r in range(L):
        i = g * L + r
        u0 = ub[i, pl.ds(0, L)] * um0
        u1 = ub[i, pl.ds(L, L)] * um1
        p0 = pb[i, pl.ds(0, L)] * pm0
        p1 = pb[i, pl.ds(L, L)] * pm1
        o = j * CH + i
        concat_v[o, pl.ds(0, L)] = u0
        concat_v[o, pl.ds(L, L)] = u1
        concat_v[o, pl.ds(2 * L, L)] = p0
        concat_v[o, pl.ds(3 * L, L)] = p1
        t = u0 * p0 + u1 * p1
        plsc.store_scatter(tbuf_v, [lane17 + r], t)
      acc = tbuf_v[pl.ds(0, L)]
      for k in range(1, L):
        acc = acc + tbuf_v[pl.ds(k * (L + 1), L)]
      og = j * CH + g * L
      score_v[pl.ds(og, L)] = acc + bias_v[pl.ds(og, L)]
      return carry

    lax.fori_loop(0, CH // L, group, 0)

  pltpu.sync_copy(score_v, score_hbm.at[pl.ds(base, BPW)])
  pltpu.sync_copy(concat_v, concat_hbm.at[pl.ds(base, BPW)])


@jax.jit
def _mvem_sc(uidx, pidx, user_emb, product_emb, product_bias,
             user_emb_mask, product_emb_mask):
  mesh = plsc.VectorSubcoreMesh(
      core_axis_name="c", subcore_axis_name="s", num_cores=NC,
      num_subcores=NS)
  run = pl.kernel(
      _sc_body,
      out_type=(jax.ShapeDtypeStruct((B,), jnp.float32),
                jax.ShapeDtypeStruct((B, 2 * D), jnp.float32)),
      mesh=mesh,
      scratch_types=[
          pltpu.VMEM((BPW,), jnp.int32),
          pltpu.VMEM((BPW,), jnp.int32),
          pltpu.VMEM((2, CH, D), jnp.float32),
          pltpu.VMEM((2, CH, D), jnp.float32),
          pltpu.VMEM((BPW,), jnp.float32),
          pltpu.VMEM((D,), jnp.float32),
          pltpu.VMEM((D,), jnp.float32),
          pltpu.VMEM((BPW, 2 * D), jnp.float32),
          pltpu.VMEM((BPW,), jnp.float32),
          pltpu.VMEM((L * (L + 1),), jnp.float32),
          pltpu.SemaphoreType.DMA,
          pltpu.SemaphoreType.DMA,
          pltpu.SemaphoreType.DMA,
      ],
      compiler_params=pltpu.CompilerParams(
          needs_layout_passes=False, use_tc_tiling_on_sc=True),
  )
  return run(uidx, pidx, user_emb, product_emb, product_bias,
             user_emb_mask, product_emb_mask)


def kernel(user_idx, product_idx, user_emb, product_emb, product_bias,
           user_emb_mask, product_emb_mask):
  score, concat = _mvem_sc(user_idx.astype(jnp.int32),
                           product_idx.astype(jnp.int32),
                           user_emb, product_emb, product_bias,
                           user_emb_mask, product_emb_mask)
  return score, concat


# speedup vs baseline: 1.4781x; 1.0006x over previous
import jax
import jax.numpy as jnp
from jax import lax
from jax.experimental import pallas as pl
from jax.experimental.pallas import tpu as pltpu
from jax.experimental.pallas import tpu_sc as plsc

B = 16384
D = 32
NC = 2
NS = 16
NW = NC * NS
BPW = B // NW
L = 16
NCH = 8
CH = BPW // NCH


def _sc_body(uidx_hbm, pidx_hbm, uemb_hbm, pemb_hbm, pbias_hbm,
             umask_hbm, pmask_hbm,
             score_hbm, concat_hbm,
             uidx_v, pidx_v, ubuf_v, pbuf_v,
             bias_v, umask_v, pmask_v, concat_v, score_v, tbuf_v,
             sem_u, sem_p, sem_b):
  c = lax.axis_index("c")
  s = lax.axis_index("s")
  wid = s * NC + c
  base = wid * BPW

  pltpu.sync_copy(uidx_hbm.at[pl.ds(base, BPW)], uidx_v)
  pltpu.sync_copy(pidx_hbm.at[pl.ds(base, BPW)], pidx_v)
  pltpu.sync_copy(umask_hbm, umask_v)
  pltpu.sync_copy(pmask_hbm, pmask_v)

  bias_cp = pltpu.async_copy(pbias_hbm.at[pidx_v], bias_v, sem_b)

  lane = lax.iota(jnp.int32, L)
  zero16 = jnp.zeros((L,), jnp.int32)

  def fire(j):
    @plsc.parallel_loop(0, CH, unroll=4)
    def row_fetch(i):
      off = j * CH + (i // L) * L
      sel = lane == (i % L)
      ur = jnp.sum(jnp.where(sel, uidx_v[pl.ds(off, L)], zero16))
      pr = jnp.sum(jnp.where(sel, pidx_v[pl.ds(off, L)], zero16))
      pltpu.async_copy(uemb_hbm.at[ur], ubuf_v.at[j % 2, i], sem_u)
      pltpu.async_copy(pemb_hbm.at[pr], pbuf_v.at[j % 2, i], sem_p)

  def drain(j):
    pltpu.make_async_copy(uemb_hbm.at[pl.ds(0, CH)], ubuf_v.at[j % 2],
                          sem_u).wait()
    pltpu.make_async_copy(pemb_hbm.at[pl.ds(0, CH)], pbuf_v.at[j % 2],
                          sem_p).wait()

  um0 = umask_v[pl.ds(0, L)]
  um1 = umask_v[pl.ds(L, L)]
  pm0 = pmask_v[pl.ds(0, L)]
  pm1 = pmask_v[pl.ds(L, L)]
  lane17 = lax.iota(jnp.int32, L) * (L + 1)

  fire(0)
  bias_cp.wait()
  for j in range(NCH):
    if j + 1 < NCH:
      fire(j + 1)
    drain(j)
    ub = ubuf_v.at[j % 2]
    pb = pbuf_v.at[j % 2]

    def group(g, carry, ub=ub, pb=pb, j=j):
      for r in range(L):
        i = g * L + r
        u0 = ub[i, pl.ds(0, L)] * um0
        u1 = ub[i, pl.ds(L, L)] * um1
        p0 = pb[i, pl.ds(0, L)] * pm0
        p1 = pb[i, pl.ds(L, L)] * pm1
        o = j * CH + i
        concat_v[o, pl.ds(0, L)] = u0
        concat_v[o, pl.ds(L, L)] = u1
        concat_v[o, pl.ds(2 * L, L)] = p0
        concat_v[o, pl.ds(3 * L, L)] = p1
        t = u0 * p0 + u1 * p1
        plsc.store_scatter(tbuf_v, [lane17 + r], t)
      acc = tbuf_v[pl.ds(0, L)]
      for k in range(1, L):
        acc = acc + tbuf_v[pl.ds(k * (L + 1), L)]
      og = j * CH + g * L
      score_v[pl.ds(og, L)] = acc + bias_v[pl.ds(og, L)]
      return carry

    lax.fori_loop(0, CH // L, group, 0)

  pltpu.sync_copy(score_v, score_hbm.at[pl.ds(base, BPW)])
  pltpu.sync_copy(concat_v, concat_hbm.at[pl.ds(base, BPW)])


@jax.jit
def _mvem_sc(uidx, pidx, user_emb, product_emb, product_bias,
             user_emb_mask, product_emb_mask):
  mesh = plsc.VectorSubcoreMesh(
      core_axis_name="c", subcore_axis_name="s", num_cores=NC,
      num_subcores=NS)
  run = pl.kernel(
      _sc_body,
      out_type=(jax.ShapeDtypeStruct((B,), jnp.float32),
                jax.ShapeDtypeStruct((B, 2 * D), jnp.float32)),
      mesh=mesh,
      scratch_types=[
          pltpu.VMEM((BPW,), jnp.int32),
          pltpu.VMEM((BPW,), jnp.int32),
          pltpu.VMEM((2, CH, D), jnp.float32),
          pltpu.VMEM((2, CH, D), jnp.float32),
          pltpu.VMEM((BPW,), jnp.float32),
          pltpu.VMEM((D,), jnp.float32),
          pltpu.VMEM((D,), jnp.float32),
          pltpu.VMEM((BPW, 2 * D), jnp.float32),
          pltpu.VMEM((BPW,), jnp.float32),
          pltpu.VMEM((L * (L + 1),), jnp.float32),
          pltpu.SemaphoreType.DMA,
          pltpu.SemaphoreType.DMA,
          pltpu.SemaphoreType.DMA,
      ],
      compiler_params=pltpu.CompilerParams(
          needs_layout_passes=False, use_tc_tiling_on_sc=True),
  )
  return run(uidx, pidx, user_emb, product_emb, product_bias,
             user_emb_mask, product_emb_mask)


def kernel(user_idx, product_idx, user_emb, product_emb, product_bias,
           user_emb_mask, product_emb_mask):
  score, concat = _mvem_sc(user_idx.astype(jnp.int32),
                           product_idx.astype(jnp.int32),
                           user_emb, product_emb, product_bias,
                           user_emb_mask, product_emb_mask)
  return score, concat
